# baseline (speedup 1.0000x reference)
import functools

import jax
import jax.numpy as jnp
from jax import lax
from jax.experimental import pallas as pl
from jax.experimental.pallas import tpu as pltpu
from jax.experimental.pallas import tpu_sc as plsc

_EPS = 1e-8

_N, _D = 65536, 32
_DW = _D // 2
_P = 983040

_NW = 32
_CH_PAIRS = 512
_CH_ROWS = 2 * _CH_PAIRS
_GATHERS = _CH_PAIRS // 128
_PAIRS_PER_W = _P // _NW
_CHUNKS = _PAIRS_PER_W // _CH_PAIRS
_RING = 2


def _sc_pair_d2(emb_pk, pos_pk, neg_pk):
    mesh = plsc.VectorSubcoreMesh(core_axis_name="c", subcore_axis_name="s")

    @functools.partial(
        pl.kernel,
        mesh=mesh,
        out_type=[
            jax.ShapeDtypeStruct((_P,), jnp.bfloat16),
            jax.ShapeDtypeStruct((_P,), jnp.bfloat16),
        ],
        scratch_types=[
            pltpu.VMEM((_RING, _CH_PAIRS), jnp.int32),
            pltpu.VMEM((_RING, 2, _CH_PAIRS), jnp.int32),
            pltpu.VMEM((_RING, _CH_ROWS, _DW), jnp.int32),
            pltpu.VMEM((_PAIRS_PER_W,), jnp.bfloat16),
            pltpu.VMEM_SHARED((_N, _DW), jnp.int32),
            pltpu.SemaphoreType.DMA,
            pltpu.SemaphoreType.DMA,
        ],
        compiler_params=pltpu.CompilerParams(
            needs_layout_passes=False, use_tc_tiling_on_sc=False),
    )
    def k(emb_hbm, pos_hbm, neg_hbm, dpos_hbm, dneg_hbm,
          pk_v, idx_v, rows_v, d2_v, tab_sh, gsem0, gsem1):
        wid = lax.axis_index("s") * 2 + lax.axis_index("c")
        pair_base = wid * _PAIRS_PER_W
        gsems = (gsem0, gsem1)

        @pl.when(lax.axis_index("s") == 0)
        def _load_table():
            pltpu.sync_copy(emb_hbm, tab_sh)

        plsc.subcore_barrier()

        for idx_hbm, out_hbm in ((pos_hbm, dpos_hbm), (neg_hbm, dneg_hbm)):

            def fill(c, b, idx_hbm=idx_hbm):
                pltpu.sync_copy(
                    idx_hbm.at[pl.ds(pair_base + c * _CH_PAIRS, _CH_PAIRS)],
                    pk_v.at[b],
                )
                for g in range(_CH_PAIRS // 16):
                    p = pk_v.at[b][pl.ds(g * 16, 16)]
                    idx_v.at[b].at[0][pl.ds(g * 16, 16)] = p & 0xFFFF
                    idx_v.at[b].at[1][pl.ds(g * 16, 16)] = (
                        lax.shift_right_logical(p, 16))
                for h in range(2):
                    for g in range(_GATHERS):
                        pltpu.async_copy(
                            tab_sh.at[idx_v.at[b].at[h]
                                      .at[pl.ds(g * 128, 128)]],
                            rows_v.at[b].at[pl.ds(h * _CH_PAIRS + g * 128,
                                                  128)],
                            gsems[b],
                        )

            def drain_gathers(b):
                for g in range(2 * _GATHERS):
                    pltpu.make_async_copy(
                        emb_hbm.at[pl.ds(0, 128)],
                        rows_v.at[b].at[pl.ds(g * 128, 128)],
                        gsems[b],
                    ).wait()

            def compute(c, b):
                rows = rows_v.at[b]

                def halfsum(p0):
                    ri = p0 + lax.iota(jnp.int32, 16)
                    rj = ri + _CH_PAIRS
                    acc0 = jnp.zeros((16,), jnp.float32)
                    acc1 = jnp.zeros((16,), jnp.float32)
                    for dh in range(_DW):
                        dd = jnp.full((16,), dh, jnp.int32)
                        gi = plsc.load_gather(rows, [ri, dd])
                        gj = plsc.load_gather(rows, [rj, dd])
                        ai, bi = plsc.unpack(
                            plsc.bitcast(gi, jnp.bfloat16),
                            format=plsc.PackFormat.INTERLEAVED)
                        aj, bj = plsc.unpack(
                            plsc.bitcast(gj, jnp.bfloat16),
                            format=plsc.PackFormat.INTERLEAVED)
                        t0 = ai - aj
                        t1 = bi - bj
                        acc0 = acc0 + t0 * t0
                        acc1 = acc1 + t1 * t1
                    return acc0 + acc1

                def group_body(i, carry2):
                    p0 = i * 32
                    lo = halfsum(p0)
                    hi = halfsum(p0 + 16)
                    d2_v[pl.ds(c * _CH_PAIRS + p0, 32)] = plsc.pack(
                        lo, hi, format=plsc.PackFormat.INTERLEAVED)
                    return carry2

                lax.fori_loop(0, _CH_PAIRS // 32, group_body, 0)

            for b in range(_RING):
                fill(b, b)

            def ring_body(c2, carry):
                for b in range(_RING):
                    c = c2 * _RING + b
                    drain_gathers(b)
                    compute(c, b)
                    fill(c + _RING, b)
                return carry

            lax.fori_loop(0, _CHUNKS // _RING - 1, ring_body, 0)

            for b in range(_RING):
                c = _CHUNKS - _RING + b
                drain_gathers(b)
                compute(c, b)

            pltpu.sync_copy(
                d2_v, out_hbm.at[pl.ds(pair_base, _PAIRS_PER_W)])

    return k(emb_pk, pos_pk, neg_pk)


_TC_BLK = 65536
_TC_GRID = _P // _TC_BLK


def _tc_reduce(dpos, dneg):
    def body(pos_ref, neg_ref, out_ref):
        @pl.when(pl.program_id(0) == 0)
        def _init():
            out_ref[0, 0] = 0.0

        pos_term = jnp.log1p(pos_ref[...].astype(jnp.float32) + _EPS)
        d = neg_ref[...].astype(jnp.float32) + _EPS
        q = 1.0 / (1.0 + d)
        neg_term = -jnp.log(1.0 - q + _EPS)
        out_ref[0, 0] += jnp.sum(pos_term) + jnp.sum(neg_term)

    out = pl.pallas_call(
        body,
        grid=(_TC_GRID,),
        in_specs=[
            pl.BlockSpec((_TC_BLK,), lambda i: (i,)),
            pl.BlockSpec((_TC_BLK,), lambda i: (i,)),
        ],
        out_specs=pl.BlockSpec(memory_space=pltpu.SMEM),
        out_shape=jax.ShapeDtypeStruct((1, 1), jnp.float32),
    )(dpos, dneg)
    return out[0, 0] / jnp.float32(_P)


def _pack_pairs(idx):
    idx = idx.astype(jnp.int32)
    return idx[:, 0] | (idx[:, 1] << 16)


def _pack_emb(embeddings):
    bf = embeddings.astype(jnp.bfloat16).reshape(_N, _DW, 2)
    return lax.bitcast_convert_type(bf, jnp.int32)


def kernel(embeddings, batch_pos_indices, batch_neg_indices):
    dpos, dneg = _sc_pair_d2(
        _pack_emb(embeddings),
        _pack_pairs(batch_pos_indices),
        _pack_pairs(batch_neg_indices))
    return _tc_reduce(dpos, dneg)


# speedup vs baseline: 1.1285x; 1.1285x over previous
import functools

import jax
import jax.numpy as jnp
from jax import lax
from jax.experimental import pallas as pl
from jax.experimental.pallas import tpu as pltpu
from jax.experimental.pallas import tpu_sc as plsc

_EPS = 1e-8

_N, _D = 65536, 32
_DW = _D // 2
_P = 983040

_NW = 32
_CH_PAIRS = 512
_CH_ROWS = 2 * _CH_PAIRS
_GATHERS = _CH_PAIRS // 128
_PAIRS_PER_W = _P // _NW
_CHUNKS = _PAIRS_PER_W // _CH_PAIRS
_RING = 3


def _sc_pair_d2(emb_pk, pos_pk, neg_pk):
    mesh = plsc.VectorSubcoreMesh(core_axis_name="c", subcore_axis_name="s")

    @functools.partial(
        pl.kernel,
        mesh=mesh,
        out_type=[
            jax.ShapeDtypeStruct((_P,), jnp.float32),
            jax.ShapeDtypeStruct((_P,), jnp.float32),
        ],
        scratch_types=[
            pltpu.VMEM((_PAIRS_PER_W,), jnp.int32),
            pltpu.VMEM((_RING, 2, _CH_PAIRS), jnp.int32),
            pltpu.VMEM((_RING, _CH_ROWS, _DW), jnp.int32),
            pltpu.VMEM((_PAIRS_PER_W,), jnp.float32),
            pltpu.SemaphoreType.DMA,
            pltpu.SemaphoreType.DMA,
            pltpu.SemaphoreType.DMA,
        ],
        compiler_params=pltpu.CompilerParams(
            needs_layout_passes=False, use_tc_tiling_on_sc=False),
    )
    def k(emb_hbm, pos_hbm, neg_hbm, dpos_hbm, dneg_hbm,
          pk_v, idx_v, rows_v, d2_v, gsem0, gsem1, gsem2):
        wid = lax.axis_index("s") * 2 + lax.axis_index("c")
        pair_base = wid * _PAIRS_PER_W
        gsems = (gsem0, gsem1, gsem2)

        for idx_hbm, out_hbm in ((pos_hbm, dpos_hbm), (neg_hbm, dneg_hbm)):
            pltpu.sync_copy(
                idx_hbm.at[pl.ds(pair_base, _PAIRS_PER_W)], pk_v)

            def fill(c, b):
                for g in range(_CH_PAIRS // 16):
                    p = pk_v[pl.ds(c * _CH_PAIRS + g * 16, 16)]
                    idx_v.at[b].at[0][pl.ds(g * 16, 16)] = p & 0xFFFF
                    idx_v.at[b].at[1][pl.ds(g * 16, 16)] = (
                        lax.shift_right_logical(p, 16))
                for h in range(2):
                    for g in range(_GATHERS):
                        pltpu.async_copy(
                            emb_hbm.at[idx_v.at[b].at[h]
                                       .at[pl.ds(g * 128, 128)]],
                            rows_v.at[b].at[pl.ds(h * _CH_PAIRS + g * 128,
                                                  128)],
                            gsems[b],
                        )

            def drain_gathers(b):
                for g in range(2 * _GATHERS):
                    pltpu.make_async_copy(
                        emb_hbm.at[pl.ds(0, 128)],
                        rows_v.at[b].at[pl.ds(g * 128, 128)],
                        gsems[b],
                    ).wait()

            def compute(c, b):
                rows = rows_v.at[b]

                def group_body(i, carry2):
                    p0 = i * 16
                    ri = p0 + lax.iota(jnp.int32, 16)
                    rj = ri + _CH_PAIRS
                    accs = [jnp.zeros((32,), jnp.bfloat16) for _ in range(2)]
                    for dh in range(_DW):
                        dd = jnp.full((16,), dh, jnp.int32)
                        gi = plsc.load_gather(rows, [ri, dd])
                        gj = plsc.load_gather(rows, [rj, dd])
                        t = (plsc.bitcast(gi, jnp.bfloat16)
                             - plsc.bitcast(gj, jnp.bfloat16))
                        accs[dh & 1] = accs[dh & 1] + t * t
                    lo0, hi0 = plsc.unpack(
                        accs[0], format=plsc.PackFormat.INTERLEAVED)
                    lo1, hi1 = plsc.unpack(
                        accs[1], format=plsc.PackFormat.INTERLEAVED)
                    d2_v[pl.ds(c * _CH_PAIRS + p0, 16)] = (
                        (lo0 + hi0) + (lo1 + hi1))
                    return carry2

                lax.fori_loop(0, _CH_PAIRS // 16, group_body, 0)

            for b in range(_RING):
                fill(b, b)

            def ring_body(c3, carry):
                for b in range(_RING):
                    c = c3 * _RING + b
                    drain_gathers(b)
                    compute(c, b)
                    fill(c + _RING, b)
                return carry

            lax.fori_loop(0, _CHUNKS // _RING - 1, ring_body, 0)

            for b in range(_RING):
                c = _CHUNKS - _RING + b
                drain_gathers(b)
                compute(c, b)

            pltpu.sync_copy(
                d2_v, out_hbm.at[pl.ds(pair_base, _PAIRS_PER_W)])

    return k(emb_pk, pos_pk, neg_pk)


_TC_BLK = 65536
_TC_GRID = _P // _TC_BLK


def _tc_reduce(dpos, dneg):
    def body(pos_ref, neg_ref, out_ref):
        @pl.when(pl.program_id(0) == 0)
        def _init():
            out_ref[0, 0] = 0.0

        pos_term = jnp.log1p(pos_ref[...] + _EPS)
        d = neg_ref[...] + _EPS
        q = 1.0 / (1.0 + d)
        neg_term = -jnp.log(1.0 - q + _EPS)
        out_ref[0, 0] += jnp.sum(pos_term) + jnp.sum(neg_term)

    out = pl.pallas_call(
        body,
        grid=(_TC_GRID,),
        in_specs=[
            pl.BlockSpec((_TC_BLK,), lambda i: (i,)),
            pl.BlockSpec((_TC_BLK,), lambda i: (i,)),
        ],
        out_specs=pl.BlockSpec(memory_space=pltpu.SMEM),
        out_shape=jax.ShapeDtypeStruct((1, 1), jnp.float32),
    )(dpos, dneg)
    return out[0, 0] / jnp.float32(_P)


def _pack_pairs(idx):
    idx = idx.astype(jnp.int32)
    return idx[:, 0] | (idx[:, 1] << 16)


def _pack_emb(embeddings):
    bf = embeddings.astype(jnp.bfloat16).reshape(_N, _DW, 2)
    return lax.bitcast_convert_type(bf, jnp.int32)


def kernel(embeddings, batch_pos_indices, batch_neg_indices):
    dpos, dneg = _sc_pair_d2(
        _pack_emb(embeddings),
        _pack_pairs(batch_pos_indices),
        _pack_pairs(batch_neg_indices))
    return _tc_reduce(dpos, dneg)


# speedup vs baseline: 1.2718x; 1.1270x over previous
import functools

import jax
import jax.numpy as jnp
from jax import lax
from jax.experimental import pallas as pl
from jax.experimental.pallas import tpu as pltpu
from jax.experimental.pallas import tpu_sc as plsc

_EPS = 1e-8

_N, _D = 65536, 32
_DW = _D // 2
_P = 983040

_NW = 32
_CH_PAIRS = 512
_CH_ROWS = 2 * _CH_PAIRS
_GATHERS = _CH_PAIRS // 128
_PAIRS_PER_W = _P // _NW
_CHUNKS = _PAIRS_PER_W // _CH_PAIRS
_RING = 2


def _sc_pair_d2(emb_pk, pos_pk, neg_pk):
    mesh = plsc.VectorSubcoreMesh(core_axis_name="c", subcore_axis_name="s")

    @functools.partial(
        pl.kernel,
        mesh=mesh,
        out_type=[
            jax.ShapeDtypeStruct((_P,), jnp.bfloat16),
            jax.ShapeDtypeStruct((_P,), jnp.bfloat16),
        ],
        scratch_types=[
            pltpu.VMEM((_RING, _CH_PAIRS), jnp.int32),
            pltpu.VMEM((_RING, 2, _CH_PAIRS), jnp.int32),
            pltpu.VMEM((_RING, _CH_ROWS, _DW), jnp.int32),
            pltpu.VMEM((_PAIRS_PER_W,), jnp.bfloat16),
            pltpu.VMEM_SHARED((_N, _DW), jnp.int32),
            pltpu.SemaphoreType.DMA,
            pltpu.SemaphoreType.DMA,
            pltpu.SemaphoreType.DMA,
            pltpu.SemaphoreType.DMA,
        ],
        compiler_params=pltpu.CompilerParams(
            needs_layout_passes=False, use_tc_tiling_on_sc=False),
    )
    def k(emb_hbm, pos_hbm, neg_hbm, dpos_hbm, dneg_hbm,
          pk_v, idx_v, rows_v, d2_v, tab_sh, gsem0, gsem1, psem0, psem1):
        wid = lax.axis_index("s") * 2 + lax.axis_index("c")
        pair_base = wid * _PAIRS_PER_W
        gsems = (gsem0, gsem1)
        psems = (psem0, psem1)

        @pl.when(lax.axis_index("s") == 0)
        def _load_table():
            pltpu.sync_copy(emb_hbm, tab_sh)

        plsc.subcore_barrier()

        for idx_hbm, out_hbm in ((pos_hbm, dpos_hbm), (neg_hbm, dneg_hbm)):

            def stage_pairs(c, b, idx_hbm=idx_hbm):
                pltpu.async_copy(
                    idx_hbm.at[pl.ds(pair_base + c * _CH_PAIRS, _CH_PAIRS)],
                    pk_v.at[b],
                    psems[b],
                )

            def fill(c, b, idx_hbm=idx_hbm):
                pltpu.make_async_copy(
                    idx_hbm.at[pl.ds(0, _CH_PAIRS)], pk_v.at[b], psems[b]
                ).wait()
                for g in range(_CH_PAIRS // 16):
                    p = pk_v.at[b][pl.ds(g * 16, 16)]
                    idx_v.at[b].at[0][pl.ds(g * 16, 16)] = p & 0xFFFF
                    idx_v.at[b].at[1][pl.ds(g * 16, 16)] = (
                        lax.shift_right_logical(p, 16))
                for h in range(2):
                    for g in range(_GATHERS):
                        pltpu.async_copy(
                            tab_sh.at[idx_v.at[b].at[h]
                                      .at[pl.ds(g * 128, 128)]],
                            rows_v.at[b].at[pl.ds(h * _CH_PAIRS + g * 128,
                                                  128)],
                            gsems[b],
                        )

            def drain_gathers(b):
                for g in range(2 * _GATHERS):
                    pltpu.make_async_copy(
                        emb_hbm.at[pl.ds(0, 128)],
                        rows_v.at[b].at[pl.ds(g * 128, 128)],
                        gsems[b],
                    ).wait()

            def compute(c, b):
                rows = rows_v.at[b]

                def halfsum(p0):
                    ri = p0 + lax.iota(jnp.int32, 16)
                    rj = ri + _CH_PAIRS
                    accs = [jnp.zeros((32,), jnp.bfloat16) for _ in range(2)]
                    for dh in range(_DW):
                        dd = jnp.full((16,), dh, jnp.int32)
                        gi = plsc.load_gather(rows, [ri, dd])
                        gj = plsc.load_gather(rows, [rj, dd])
                        t = (plsc.bitcast(gi, jnp.bfloat16)
                             - plsc.bitcast(gj, jnp.bfloat16))
                        accs[dh & 1] = accs[dh & 1] + t * t
                    lo0, hi0 = plsc.unpack(
                        accs[0], format=plsc.PackFormat.INTERLEAVED)
                    lo1, hi1 = plsc.unpack(
                        accs[1], format=plsc.PackFormat.INTERLEAVED)
                    return (lo0 + hi0) + (lo1 + hi1)

                def group_body(i, carry2):
                    p0 = i * 32
                    lo = halfsum(p0)
                    hi = halfsum(p0 + 16)
                    d2_v[pl.ds(c * _CH_PAIRS + p0, 32)] = plsc.pack(
                        lo, hi, format=plsc.PackFormat.INTERLEAVED)
                    return carry2

                lax.fori_loop(0, _CH_PAIRS // 32, group_body, 0)

            for b in range(_RING):
                stage_pairs(b, b)
            for b in range(_RING):
                fill(b, b)

            def ring_body(c2, carry):
                for b in range(_RING):
                    c = c2 * _RING + b
                    stage_pairs(c + _RING, b)
                    drain_gathers(b)
                    compute(c, b)
                    fill(c + _RING, b)
                return carry

            lax.fori_loop(0, _CHUNKS // _RING - 1, ring_body, 0)

            for b in range(_RING):
                c = _CHUNKS - _RING + b
                drain_gathers(b)
                compute(c, b)

            pltpu.sync_copy(
                d2_v, out_hbm.at[pl.ds(pair_base, _PAIRS_PER_W)])

    return k(emb_pk, pos_pk, neg_pk)


_TC_BLK = 65536
_TC_GRID = _P // _TC_BLK


def _tc_reduce(dpos, dneg):
    def body(pos_ref, neg_ref, out_ref):
        @pl.when(pl.program_id(0) == 0)
        def _init():
            out_ref[0, 0] = 0.0

        pos_term = jnp.log1p(pos_ref[...].astype(jnp.float32) + _EPS)
        d = neg_ref[...].astype(jnp.float32) + _EPS
        q = 1.0 / (1.0 + d)
        neg_term = -jnp.log(1.0 - q + _EPS)
        out_ref[0, 0] += jnp.sum(pos_term) + jnp.sum(neg_term)

    out = pl.pallas_call(
        body,
        grid=(_TC_GRID,),
        in_specs=[
            pl.BlockSpec((_TC_BLK,), lambda i: (i,)),
            pl.BlockSpec((_TC_BLK,), lambda i: (i,)),
        ],
        out_specs=pl.BlockSpec(memory_space=pltpu.SMEM),
        out_shape=jax.ShapeDtypeStruct((1, 1), jnp.float32),
    )(dpos, dneg)
    return out[0, 0] / jnp.float32(_P)


def _pack_pairs(idx):
    idx = idx.astype(jnp.int32)
    return idx[:, 0] | (idx[:, 1] << 16)


def _pack_emb(embeddings):
    bf = embeddings.astype(jnp.bfloat16).reshape(_N, _DW, 2)
    return lax.bitcast_convert_type(bf, jnp.int32)


def kernel(embeddings, batch_pos_indices, batch_neg_indices):
    dpos, dneg = _sc_pair_d2(
        _pack_emb(embeddings),
        _pack_pairs(batch_pos_indices),
        _pack_pairs(batch_neg_indices))
    return _tc_reduce(dpos, dneg)
